# Initial kernel scaffold; baseline (speedup 1.0000x reference)
import jax
import jax.numpy as jnp
from jax.experimental import pallas as pl


def kernel(x, edge_index, edge_weight, Wz, bz, Wr, br, Wh, bh, lz_w, lz_b, lr_w, lr_b, lh_w, lh_b, lin_w, lin_b):
    raise NotImplementedError("write your pallas kernel here")


# speedup vs baseline: 124.5626x; 124.5626x over previous
import jax
import jax.numpy as jnp
from jax import lax
from jax.experimental import pallas as pl
from jax.experimental.pallas import tpu as pltpu
from jax.experimental.pallas import tpu_sc as plsc

N = 10000
E = 320000
B = 2
T = 4
FIN = 128
HID = 64
OUT = 32

NC = 2
NS = 16
LANES = 16
G = 80
EPT = E // NS
NGRP = EPT // G
SLAB = 25
NSLAB = NGRP // SLAB
QPC = B * T // NC
NB = 1000
VR = FIN // LANES
GV = G // LANES


def _rsqrt_newton(d):
    i = lax.bitcast_convert_type(d, jnp.int32)
    i = jnp.int32(0x5F3759DF) - (i >> 1)
    y = lax.bitcast_convert_type(i, jnp.float32)
    for _ in range(3):
        y = y * (1.5 - 0.5 * d * y * y)
    return jnp.where(d > 0.0, y, 0.0)


def _norm_kernel(src_hbm, dst_hbm, w_hbm, norm_hbm, selfn_hbm,
                 dst_s, w_s, src_s, dinv_t, selfn_s, zero_b, deg_sp):
    c = lax.axis_index("c")
    s = lax.axis_index("s")

    def _zf(v, _):
        zero_b[pl.ds(v * LANES, LANES)] = jnp.zeros((LANES,), jnp.float32)
        return 0
    lax.fori_loop(0, 640 // LANES, _zf, 0)

    @pl.when(s < NS - 1)
    def _():
        pltpu.sync_copy(zero_b, deg_sp.at[pl.ds(s * 640, 640)])

    @pl.when(s == NS - 1)
    def _():
        pltpu.sync_copy(zero_b.at[pl.ds(0, 400)], deg_sp.at[pl.ds(9600, 400)])

    plsc.subcore_barrier()

    def _deg_slab(sl, _):
        pltpu.sync_copy(dst_hbm.at[s * NSLAB + sl], dst_s)
        pltpu.sync_copy(w_hbm.at[s * NSLAB + sl], w_s)

        def _deg_add(g, _):
            pltpu.sync_copy(w_s.at[g], deg_sp.at[dst_s.at[g]], add=True)
            return 0

        lax.fori_loop(0, SLAB, _deg_add, 0)
        return 0

    lax.fori_loop(0, NSLAB, _deg_slab, 0)

    plsc.subcore_barrier()

    pltpu.sync_copy(deg_sp, dinv_t)

    def _dinv(i, _):
        d = dinv_t[pl.ds(i * LANES, LANES)] + 1.0
        dinv_t[pl.ds(i * LANES, LANES)] = _rsqrt_newton(d)
        return 0

    lax.fori_loop(0, N // LANES, _dinv, 0)

    @pl.when(c == 0)
    def _():
        def _norm_slab(sl, _):
            pltpu.sync_copy(dst_hbm.at[s * NSLAB + sl], dst_s)
            pltpu.sync_copy(w_hbm.at[s * NSLAB + sl], w_s)
            pltpu.sync_copy(src_hbm.at[s * NSLAB + sl], src_s)

            def _norm_row(r, _):
                for v in range(GV):
                    vsl = pl.ds(v * LANES, LANES)
                    sv = plsc.load_gather(dinv_t, [src_s[r, vsl]])
                    dv = plsc.load_gather(dinv_t, [dst_s[r, vsl]])
                    w_s[r, vsl] = sv * w_s[r, vsl] * dv
                return 0

            lax.fori_loop(0, SLAB, _norm_row, 0)
            pltpu.sync_copy(w_s, norm_hbm.at[s * NSLAB + sl])
            return 0

        lax.fori_loop(0, NSLAB, _norm_slab, 0)

    @pl.when(jnp.logical_and(c == 0, s < N // G // SLAB))
    def _():
        def _selfn(i, _):
            r = i // GV
            v = i % GV
            dv = dinv_t[pl.ds(s * (SLAB * G) + i * LANES, LANES)]
            selfn_s[r, pl.ds(v * LANES, LANES)] = dv * dv
            return 0

        lax.fori_loop(0, SLAB * G // LANES, _selfn, 0)
        pltpu.sync_copy(selfn_s, selfn_hbm.at[s])


def _prop_kernel(x_hbm, src_hbm, dst_hbm, norm_hbm, selfn_hbm, selfid_hbm,
                 agg_hbm,
                 srcq_s, dst_s, norm_s, selfn_b, selfid_b, rows_b, acc_sp):
    c = lax.axis_index("c")
    s = lax.axis_index("s")

    nrows = jnp.where(s < NS - 1, 640, 400)
    nselfg = nrows // G
    row0 = s * 640

    pltpu.sync_copy(selfn_hbm.at[pl.ds(s * 8, 8)], selfn_b)
    pltpu.sync_copy(selfid_hbm.at[pl.ds(s * 8, 8)], selfid_b)

    for j in range(QPC):
        qbase = (c * QPC + j) * N

        def _zfill(i, _):
            r = i // VR
            rows_b[r, pl.ds((i % VR) * LANES, LANES)] = jnp.zeros(
                (LANES,), jnp.float32)
            return 0
        lax.fori_loop(0, G * VR, _zfill, 0)

        def _zcopy(k, _):
            pltpu.sync_copy(rows_b, acc_sp.at[pl.ds(row0 + k * G, G)])
            return 0
        lax.fori_loop(0, nrows // G, _zcopy, 0)

        plsc.subcore_barrier()

        def _slab(sl, _):
            pltpu.sync_copy(src_hbm.at[s * NSLAB + sl], srcq_s)
            pltpu.sync_copy(dst_hbm.at[s * NSLAB + sl], dst_s)
            pltpu.sync_copy(norm_hbm.at[s * NSLAB + sl], norm_s)

            def _shift(i, _):
                r = i // GV
                vs = pl.ds((i % GV) * LANES, LANES)
                srcq_s[r, vs] = srcq_s[r, vs] + qbase
                return 0

            lax.fori_loop(0, SLAB * GV, _shift, 0)

            def _edge(g, _):
                pltpu.sync_copy(x_hbm.at[srcq_s.at[g]], rows_b)

                def _srow(i, _):
                    nv = plsc.load_gather(
                        norm_s, [jnp.full((LANES,), g, jnp.int32),
                                 jnp.full((LANES,), i, jnp.int32)])
                    for v in range(VR):
                        vs = pl.ds(v * LANES, LANES)
                        rows_b[i, vs] = rows_b[i, vs] * nv
                    return 0

                lax.fori_loop(0, G, _srow, 0)
                pltpu.sync_copy(rows_b, acc_sp.at[dst_s.at[g]], add=True)
                return 0

            lax.fori_loop(0, SLAB, _edge, 0)
            return 0

        lax.fori_loop(0, NSLAB, _slab, 0)

        def _selfg(g, _):
            pltpu.sync_copy(x_hbm.at[pl.ds(qbase + row0 + g * G, G)], rows_b)

            def _srow(i, _):
                nv = plsc.load_gather(
                    selfn_b, [jnp.full((LANES,), g, jnp.int32),
                              jnp.full((LANES,), i, jnp.int32)])
                for v in range(VR):
                    sl = pl.ds(v * LANES, LANES)
                    rows_b[i, sl] = rows_b[i, sl] * nv
                return 0

            lax.fori_loop(0, G, _srow, 0)
            pltpu.sync_copy(rows_b, acc_sp.at[selfid_b.at[g]], add=True)
            return 0

        lax.fori_loop(0, nselfg, _selfg, 0)

        plsc.subcore_barrier()

        q = c * QPC + j

        @pl.when(s < NS - 1)
        def _():
            pltpu.sync_copy(acc_sp.at[pl.ds(row0, 640)],
                            agg_hbm.at[q, pl.ds(row0, 640)])

        @pl.when(s == NS - 1)
        def _():
            pltpu.sync_copy(acc_sp.at[pl.ds(9600, 400)],
                            agg_hbm.at[q, pl.ds(9600, 400)])


def _dense_kernel(agg_ref, Wz, bz, Wr, br, Wh, bh, lzw, lzb, lrw, lrb,
                  lhw, lhb, lw, lb, out_ref):
    f32 = jnp.float32
    hp = jax.lax.Precision.HIGHEST

    def mm(a, w):
        return jax.lax.dot_general(a, w, (((1,), (0,)), ((), ())),
                                   precision=hp, preferred_element_type=f32)

    H = jnp.zeros((NB, HID), f32)
    for t in range(T):
        a = agg_ref[0, t]
        cz = mm(a, Wz[...]) + bz[...]
        cr = mm(a, Wr[...]) + br[...]
        ch = mm(a, Wh[...]) + bh[...]
        Z = jax.nn.sigmoid(mm(cz, lzw[0:HID, :]) + mm(H, lzw[HID:, :])
                           + lzb[...])
        R = jax.nn.sigmoid(mm(cr, lrw[0:HID, :]) + mm(H, lrw[HID:, :])
                           + lrb[...])
        Ht = jnp.tanh(mm(ch, lhw[0:HID, :]) + mm(H * R, lhw[HID:, :])
                      + lhb[...])
        H = Z * H + (1.0 - Z) * Ht
        out_ref[0, t] = mm(jax.nn.relu(H), lw[...]) + lb[...]


def kernel(x, edge_index, edge_weight, Wz, bz, Wr, br, Wh, bh,
           lz_w, lz_b, lr_w, lr_b, lh_w, lh_b, lin_w, lin_b):
    f32 = jnp.float32
    mesh = plsc.VectorSubcoreMesh(core_axis_name="c", subcore_axis_name="s",
                                  num_cores=NC, num_subcores=NS)
    sc_params = pltpu.CompilerParams(needs_layout_passes=False)

    src = edge_index[0].astype(jnp.int32).reshape(NS * NSLAB, SLAB, G)
    dst = edge_index[1].astype(jnp.int32).reshape(NS * NSLAB, SLAB, G)
    w_r = edge_weight.astype(f32).reshape(NS * NSLAB, SLAB, G)

    k1 = pl.kernel(
        _norm_kernel,
        out_type=(
            jax.ShapeDtypeStruct((NS * NSLAB, SLAB, G), f32),
            jax.ShapeDtypeStruct((N // G // SLAB, SLAB, G), f32),
        ),
        mesh=mesh,
        scratch_types=(
            pltpu.VMEM((SLAB, G), jnp.int32),
            pltpu.VMEM((SLAB, G), f32),
            pltpu.VMEM((SLAB, G), jnp.int32),
            pltpu.VMEM((N,), f32),
            pltpu.VMEM((SLAB, G), f32),
            pltpu.VMEM((640,), f32),
            pltpu.VMEM_SHARED((N,), f32),
        ),
        compiler_params=sc_params,
    )
    norm_r, selfn_r = k1(src, dst, w_r)

    x_bt = jnp.transpose(x, (0, 3, 1, 2)).reshape(B * T * N, FIN)
    selfn_p = jnp.pad(selfn_r.reshape(N // G, G), ((0, 3), (0, 0)))
    selfid = jnp.minimum(jnp.arange(NS * 8 * G, dtype=jnp.int32),
                         N - 1).reshape(NS * 8, G)

    k2 = pl.kernel(
        _prop_kernel,
        out_type=jax.ShapeDtypeStruct((B * T, N, FIN), f32),
        mesh=mesh,
        scratch_types=(
            pltpu.VMEM((SLAB, G), jnp.int32),
            pltpu.VMEM((SLAB, G), jnp.int32),
            pltpu.VMEM((SLAB, G), f32),
            pltpu.VMEM((8, G), f32),
            pltpu.VMEM((8, G), jnp.int32),
            pltpu.VMEM((G, FIN), f32),
            pltpu.VMEM_SHARED((N, FIN), f32),
        ),
        compiler_params=sc_params,
    )
    agg = k2(x_bt, src, dst, norm_r, selfn_p, selfid)
    agg4 = agg.reshape(B, T, N, FIN)

    def wspec(a):
        return pl.BlockSpec(a.shape, lambda b, i, _n=a.ndim: (0,) * _n)

    bz2, br2, bh2 = (v.reshape(1, HID) for v in (bz, br, bh))
    lzb2, lrb2, lhb2 = (v.reshape(1, HID) for v in (lz_b, lr_b, lh_b))
    lb2 = lin_b.reshape(1, OUT)
    wargs = (Wz, bz2, Wr, br2, Wh, bh2, lz_w, lzb2, lr_w, lrb2,
             lh_w, lhb2, lin_w, lb2)

    out = pl.pallas_call(
        _dense_kernel,
        grid=(B, N // NB),
        in_specs=[pl.BlockSpec((1, T, NB, FIN), lambda b, i: (b, 0, i, 0))]
                 + [wspec(a) for a in wargs],
        out_specs=pl.BlockSpec((1, T, NB, OUT), lambda b, i: (b, 0, i, 0)),
        out_shape=jax.ShapeDtypeStruct((B, T, N, OUT), f32),
    )(agg4, *wargs)
    return out


# speedup vs baseline: 182.1052x; 1.4620x over previous
import jax
import jax.numpy as jnp
from jax import lax
from jax.experimental import pallas as pl
from jax.experimental.pallas import tpu as pltpu
from jax.experimental.pallas import tpu_sc as plsc

N = 10000
E = 320000
B = 2
T = 4
FIN = 128
HID = 64
OUT = 32

NC = 2
NS = 16
LANES = 16
G = 80
EPT = E // NS
NGRP = EPT // G
SLAB = 50
NSLAB = NGRP // SLAB
SSLAB = 25
QPC = B * T // NC
NB = 1000
VR = FIN // LANES
GV = G // LANES


def _rsqrt_newton(d):
    i = lax.bitcast_convert_type(d, jnp.int32)
    i = jnp.int32(0x5F3759DF) - (i >> 1)
    y = lax.bitcast_convert_type(i, jnp.float32)
    for _ in range(3):
        y = y * (1.5 - 0.5 * d * y * y)
    return jnp.where(d > 0.0, y, 0.0)


def _norm_kernel(src_hbm, dst_hbm, w_hbm, norm_hbm, selfn_hbm,
                 dst_s, w_s, src_s, dinv_t, selfn_s, zero_b, deg_sp):
    c = lax.axis_index("c")
    s = lax.axis_index("s")

    def _zf(v, _):
        zero_b[pl.ds(v * LANES, LANES)] = jnp.zeros((LANES,), jnp.float32)
        return 0
    lax.fori_loop(0, 640 // LANES, _zf, 0)

    @pl.when(s < NS - 1)
    def _():
        pltpu.sync_copy(zero_b, deg_sp.at[pl.ds(s * 640, 640)])

    @pl.when(s == NS - 1)
    def _():
        pltpu.sync_copy(zero_b.at[pl.ds(0, 400)], deg_sp.at[pl.ds(9600, 400)])

    plsc.subcore_barrier()

    def _deg_slab(sl, _):
        pltpu.sync_copy(dst_hbm.at[s * NSLAB + sl], dst_s)
        pltpu.sync_copy(w_hbm.at[s * NSLAB + sl], w_s)

        def _deg_add(g, _):
            pltpu.sync_copy(w_s.at[g], deg_sp.at[dst_s.at[g]], add=True)
            return 0

        lax.fori_loop(0, SLAB, _deg_add, 0)
        return 0

    lax.fori_loop(0, NSLAB, _deg_slab, 0)

    plsc.subcore_barrier()

    pltpu.sync_copy(deg_sp, dinv_t)

    def _dinv(i, _):
        d = dinv_t[pl.ds(i * LANES, LANES)] + 1.0
        dinv_t[pl.ds(i * LANES, LANES)] = _rsqrt_newton(d)
        return 0

    lax.fori_loop(0, N // LANES, _dinv, 0)

    @pl.when(c == 0)
    def _():
        def _norm_slab(sl, _):
            pltpu.sync_copy(dst_hbm.at[s * NSLAB + sl], dst_s)
            pltpu.sync_copy(w_hbm.at[s * NSLAB + sl], w_s)
            pltpu.sync_copy(src_hbm.at[s * NSLAB + sl], src_s)

            def _norm_row(r, _):
                for v in range(GV):
                    vsl = pl.ds(v * LANES, LANES)
                    sv = plsc.load_gather(dinv_t, [src_s[r, vsl]])
                    dv = plsc.load_gather(dinv_t, [dst_s[r, vsl]])
                    w_s[r, vsl] = sv * w_s[r, vsl] * dv
                return 0

            lax.fori_loop(0, SLAB, _norm_row, 0)
            pltpu.sync_copy(w_s, norm_hbm.at[s * NSLAB + sl])
            return 0

        lax.fori_loop(0, NSLAB, _norm_slab, 0)

    @pl.when(jnp.logical_and(c == 0, s < N // G // SSLAB))
    def _():
        def _selfn(i, _):
            r = i // GV
            v = i % GV
            dv = dinv_t[pl.ds(s * (SSLAB * G) + i * LANES, LANES)]
            selfn_s[r, pl.ds(v * LANES, LANES)] = dv * dv
            return 0

        lax.fori_loop(0, SSLAB * G // LANES, _selfn, 0)
        pltpu.sync_copy(selfn_s, selfn_hbm.at[s])


def _prop_kernel(x_hbm, src_hbm, dst_hbm, norm_hbm, selfn_hbm, selfid_hbm,
                 agg_hbm,
                 srcq_s, dst_s, norm_s, selfn_b, selfid_b, rows0_b, rows1_b,
                 gsem0, gsem1, ssem0, ssem1, acc_sp):
    rows = (rows0_b, rows1_b)
    gsem = (gsem0, gsem1)
    ssem = (ssem0, ssem1)
    rows_b = rows0_b
    c = lax.axis_index("c")
    s = lax.axis_index("s")

    nrows = jnp.where(s < NS - 1, 640, 400)
    nselfg = nrows // G
    row0 = s * 640

    pltpu.sync_copy(selfn_hbm.at[pl.ds(s * 8, 8)], selfn_b)
    pltpu.sync_copy(selfid_hbm.at[pl.ds(s * 8, 8)], selfid_b)

    for j in range(QPC):
        qbase = (c * QPC + j) * N

        def _zfill(i, _):
            r = i // VR
            rows_b[r, pl.ds((i % VR) * LANES, LANES)] = jnp.zeros(
                (LANES,), jnp.float32)
            return 0
        lax.fori_loop(0, G * VR, _zfill, 0)

        def _zcopy(k, _):
            pltpu.sync_copy(rows_b, acc_sp.at[pl.ds(row0 + k * G, G)])
            return 0
        lax.fori_loop(0, nrows // G, _zcopy, 0)

        plsc.subcore_barrier()

        def _slab(sl, _):
            pltpu.sync_copy(src_hbm.at[s * NSLAB + sl], srcq_s)
            pltpu.sync_copy(dst_hbm.at[s * NSLAB + sl], dst_s)
            pltpu.sync_copy(norm_hbm.at[s * NSLAB + sl], norm_s)

            def _shift(i, _):
                r = i // GV
                vs = pl.ds((i % GV) * LANES, LANES)
                srcq_s[r, vs] = srcq_s[r, vs] + qbase
                return 0

            lax.fori_loop(0, SLAB * GV, _shift, 0)

            def issue_gather(g, k):
                pltpu.async_copy(x_hbm.at[srcq_s.at[g]], rows[k], gsem[k])

            def wait_gather(g, k):
                pltpu.make_async_copy(x_hbm.at[srcq_s.at[g]], rows[k],
                                      gsem[k]).wait()

            def issue_scatter(g, k):
                pltpu.async_copy(rows[k], acc_sp.at[dst_s.at[g]], ssem[k],
                                 add=True)

            def wait_scatter(g, k):
                pltpu.make_async_copy(rows[k], acc_sp.at[dst_s.at[g]],
                                      ssem[k]).wait()

            def scale(g, k):
                def _srow(i, _):
                    nv = plsc.load_gather(
                        norm_s, [jnp.full((LANES,), g, jnp.int32),
                                 jnp.full((LANES,), i, jnp.int32)])
                    for v in range(VR):
                        vs = pl.ds(v * LANES, LANES)
                        rows[k][i, vs] = rows[k][i, vs] * nv
                    return 0

                lax.fori_loop(0, G, _srow, 0)

            def step(g, k, issue_next, wait_prev):
                if issue_next:
                    if wait_prev:
                        wait_scatter(g - 1, 1 - k)
                    issue_gather(g + 1, 1 - k)
                wait_gather(g, k)
                scale(g, k)
                issue_scatter(g, k)

            g0 = jnp.int32(0)
            issue_gather(g0, 0)
            step(g0, 0, True, False)

            def _pair(i, _):
                step(2 * i + 1, 1, True, True)
                step(2 * i + 2, 0, True, True)
                return 0

            lax.fori_loop(0, (SLAB - 2) // 2, _pair, 0)

            glast = jnp.int32(SLAB - 1)
            step(glast, 1, False, False)
            wait_scatter(jnp.int32(SLAB - 2), 0)
            wait_scatter(glast, 1)
            return 0

        lax.fori_loop(0, NSLAB, _slab, 0)

        def _selfg(g, _):
            pltpu.sync_copy(x_hbm.at[pl.ds(qbase + row0 + g * G, G)], rows_b)

            def _srow(i, _):
                nv = plsc.load_gather(
                    selfn_b, [jnp.full((LANES,), g, jnp.int32),
                              jnp.full((LANES,), i, jnp.int32)])
                for v in range(VR):
                    sl = pl.ds(v * LANES, LANES)
                    rows_b[i, sl] = rows_b[i, sl] * nv
                return 0

            lax.fori_loop(0, G, _srow, 0)
            pltpu.sync_copy(rows_b, acc_sp.at[selfid_b.at[g]], add=True)
            return 0

        lax.fori_loop(0, nselfg, _selfg, 0)

        plsc.subcore_barrier()

        q = c * QPC + j

        @pl.when(s < NS - 1)
        def _():
            pltpu.sync_copy(acc_sp.at[pl.ds(row0, 640)],
                            agg_hbm.at[q, pl.ds(row0, 640)])

        @pl.when(s == NS - 1)
        def _():
            pltpu.sync_copy(acc_sp.at[pl.ds(9600, 400)],
                            agg_hbm.at[q, pl.ds(9600, 400)])


def _dense_kernel(agg_ref, Wz, bz, Wr, br, Wh, bh, lzw, lzb, lrw, lrb,
                  lhw, lhb, lw, lb, out_ref):
    f32 = jnp.float32
    hp = jax.lax.Precision.HIGHEST

    def mm(a, w):
        return jax.lax.dot_general(a, w, (((1,), (0,)), ((), ())),
                                   precision=hp, preferred_element_type=f32)

    H = jnp.zeros((NB, HID), f32)
    for t in range(T):
        a = agg_ref[0, t]
        cz = mm(a, Wz[...]) + bz[...]
        cr = mm(a, Wr[...]) + br[...]
        ch = mm(a, Wh[...]) + bh[...]
        Z = jax.nn.sigmoid(mm(cz, lzw[0:HID, :]) + mm(H, lzw[HID:, :])
                           + lzb[...])
        R = jax.nn.sigmoid(mm(cr, lrw[0:HID, :]) + mm(H, lrw[HID:, :])
                           + lrb[...])
        Ht = jnp.tanh(mm(ch, lhw[0:HID, :]) + mm(H * R, lhw[HID:, :])
                      + lhb[...])
        H = Z * H + (1.0 - Z) * Ht
        out_ref[0, t] = mm(jax.nn.relu(H), lw[...]) + lb[...]


def kernel(x, edge_index, edge_weight, Wz, bz, Wr, br, Wh, bh,
           lz_w, lz_b, lr_w, lr_b, lh_w, lh_b, lin_w, lin_b):
    f32 = jnp.float32
    mesh = plsc.VectorSubcoreMesh(core_axis_name="c", subcore_axis_name="s",
                                  num_cores=NC, num_subcores=NS)
    sc_params = pltpu.CompilerParams(needs_layout_passes=False)

    src = edge_index[0].astype(jnp.int32).reshape(NS * NSLAB, SLAB, G)
    dst = edge_index[1].astype(jnp.int32).reshape(NS * NSLAB, SLAB, G)
    w_r = edge_weight.astype(f32).reshape(NS * NSLAB, SLAB, G)

    k1 = pl.kernel(
        _norm_kernel,
        out_type=(
            jax.ShapeDtypeStruct((NS * NSLAB, SLAB, G), f32),
            jax.ShapeDtypeStruct((N // G // SSLAB, SSLAB, G), f32),
        ),
        mesh=mesh,
        scratch_types=(
            pltpu.VMEM((SLAB, G), jnp.int32),
            pltpu.VMEM((SLAB, G), f32),
            pltpu.VMEM((SLAB, G), jnp.int32),
            pltpu.VMEM((N,), f32),
            pltpu.VMEM((SSLAB, G), f32),
            pltpu.VMEM((640,), f32),
            pltpu.VMEM_SHARED((N,), f32),
        ),
        compiler_params=sc_params,
    )
    norm_r, selfn_r = k1(src, dst, w_r)

    x_bt = jnp.transpose(x, (0, 3, 1, 2)).reshape(B * T * N, FIN)
    selfn_p = jnp.pad(selfn_r.reshape(N // G, G), ((0, 3), (0, 0)))
    selfid = jnp.minimum(jnp.arange(NS * 8 * G, dtype=jnp.int32),
                         N - 1).reshape(NS * 8, G)

    k2 = pl.kernel(
        _prop_kernel,
        out_type=jax.ShapeDtypeStruct((B * T, N, FIN), f32),
        mesh=mesh,
        scratch_types=(
            pltpu.VMEM((SLAB, G), jnp.int32),
            pltpu.VMEM((SLAB, G), jnp.int32),
            pltpu.VMEM((SLAB, G), f32),
            pltpu.VMEM((8, G), f32),
            pltpu.VMEM((8, G), jnp.int32),
            pltpu.VMEM((G, FIN), f32),
            pltpu.VMEM((G, FIN), f32),
            pltpu.SemaphoreType.DMA,
            pltpu.SemaphoreType.DMA,
            pltpu.SemaphoreType.DMA,
            pltpu.SemaphoreType.DMA,
            pltpu.VMEM_SHARED((N, FIN), f32),
        ),
        compiler_params=sc_params,
    )
    agg = k2(x_bt, src, dst, norm_r, selfn_p, selfid)
    agg4 = agg.reshape(B, T, N, FIN)

    def wspec(a):
        return pl.BlockSpec(a.shape, lambda b, i, _n=a.ndim: (0,) * _n)

    bz2, br2, bh2 = (v.reshape(1, HID) for v in (bz, br, bh))
    lzb2, lrb2, lhb2 = (v.reshape(1, HID) for v in (lz_b, lr_b, lh_b))
    lb2 = lin_b.reshape(1, OUT)
    wargs = (Wz, bz2, Wr, br2, Wh, bh2, lz_w, lzb2, lr_w, lrb2,
             lh_w, lhb2, lin_w, lb2)

    out = pl.pallas_call(
        _dense_kernel,
        grid=(B, N // NB),
        in_specs=[pl.BlockSpec((1, T, NB, FIN), lambda b, i: (b, 0, i, 0))]
                 + [wspec(a) for a in wargs],
        out_specs=pl.BlockSpec((1, T, NB, OUT), lambda b, i: (b, 0, i, 0)),
        out_shape=jax.ShapeDtypeStruct((B, T, N, OUT), f32),
    )(agg4, *wargs)
    return out


# speedup vs baseline: 261.7115x; 1.4371x over previous
import jax
import jax.numpy as jnp
from jax import lax
from jax.experimental import pallas as pl
from jax.experimental.pallas import tpu as pltpu
from jax.experimental.pallas import tpu_sc as plsc

N = 10000
E = 320000
B = 2
T = 4
FIN = 128
HID = 64
OUT = 32

NC = 2
NS = 16
LANES = 16
G = 80
EPT = E // NS
NGRP = EPT // G
SLAB = 50
NSLAB = NGRP // SLAB
SSLAB = 25
QPC = B * T // NC
NB = 1000
VR = FIN // LANES
GV = G // LANES


def _rsqrt_newton(d):
    i = lax.bitcast_convert_type(d, jnp.int32)
    i = jnp.int32(0x5F3759DF) - (i >> 1)
    y = lax.bitcast_convert_type(i, jnp.float32)
    for _ in range(3):
        y = y * (1.5 - 0.5 * d * y * y)
    return jnp.where(d > 0.0, y, 0.0)


def _norm_kernel(src_hbm, dst_hbm, w_hbm, norm_hbm, selfn_hbm,
                 dst_s, w_s, src_s, dinv_t, selfn_s, zero_b, deg_sp):
    c = lax.axis_index("c")
    s = lax.axis_index("s")

    def _zf(v, _):
        zero_b[pl.ds(v * LANES, LANES)] = jnp.zeros((LANES,), jnp.float32)
        return 0
    lax.fori_loop(0, 640 // LANES, _zf, 0)

    @pl.when(s < NS - 1)
    def _():
        pltpu.sync_copy(zero_b, deg_sp.at[pl.ds(s * 640, 640)])

    @pl.when(s == NS - 1)
    def _():
        pltpu.sync_copy(zero_b.at[pl.ds(0, 400)], deg_sp.at[pl.ds(9600, 400)])

    plsc.subcore_barrier()

    def _deg_slab(sl, _):
        pltpu.sync_copy(dst_hbm.at[s * NSLAB + sl], dst_s)
        pltpu.sync_copy(w_hbm.at[s * NSLAB + sl], w_s)

        def _deg_add(g, _):
            pltpu.sync_copy(w_s.at[g], deg_sp.at[dst_s.at[g]], add=True)
            return 0

        lax.fori_loop(0, SLAB, _deg_add, 0)
        return 0

    lax.fori_loop(0, NSLAB, _deg_slab, 0)

    plsc.subcore_barrier()

    pltpu.sync_copy(deg_sp, dinv_t)

    def _dinv(i, _):
        d = dinv_t[pl.ds(i * LANES, LANES)] + 1.0
        dinv_t[pl.ds(i * LANES, LANES)] = _rsqrt_newton(d)
        return 0

    lax.fori_loop(0, N // LANES, _dinv, 0)

    @pl.when(c == 0)
    def _():
        def _norm_slab(sl, _):
            pltpu.sync_copy(dst_hbm.at[s * NSLAB + sl], dst_s)
            pltpu.sync_copy(w_hbm.at[s * NSLAB + sl], w_s)
            pltpu.sync_copy(src_hbm.at[s * NSLAB + sl], src_s)

            def _norm_row(r, _):
                for v in range(GV):
                    vsl = pl.ds(v * LANES, LANES)
                    sv = plsc.load_gather(dinv_t, [src_s[r, vsl]])
                    dv = plsc.load_gather(dinv_t, [dst_s[r, vsl]])
                    w_s[r, vsl] = sv * w_s[r, vsl] * dv
                return 0

            lax.fori_loop(0, SLAB, _norm_row, 0)
            pltpu.sync_copy(w_s, norm_hbm.at[s * NSLAB + sl])
            return 0

        lax.fori_loop(0, NSLAB, _norm_slab, 0)

    @pl.when(jnp.logical_and(c == 0, s < N // G // SSLAB))
    def _():
        def _selfn(i, _):
            r = i // GV
            v = i % GV
            dv = dinv_t[pl.ds(s * (SSLAB * G) + i * LANES, LANES)]
            selfn_s[r, pl.ds(v * LANES, LANES)] = dv * dv
            return 0

        lax.fori_loop(0, SSLAB * G // LANES, _selfn, 0)
        pltpu.sync_copy(selfn_s, selfn_hbm.at[s])


def _prop_kernel(x_hbm, src_hbm, dst_hbm, norm_hbm, selfn_hbm, selfid_hbm,
                 agg_hbm,
                 srcq_s, dst_s, norm_s, selfn_b, selfid_b, rows0_b, rows1_b,
                 gsem0, gsem1, ssem0, ssem1, acc_sp):
    rows = (rows0_b, rows1_b)
    gsem = (gsem0, gsem1)
    ssem = (ssem0, ssem1)
    rows_b = rows0_b
    c = lax.axis_index("c")
    s = lax.axis_index("s")

    nrows = jnp.where(s < NS - 1, 640, 400)
    nselfg = nrows // G
    row0 = s * 640

    pltpu.sync_copy(selfn_hbm.at[pl.ds(s * 8, 8)], selfn_b)
    pltpu.sync_copy(selfid_hbm.at[pl.ds(s * 8, 8)], selfid_b)

    for j in range(QPC):
        qbase = (c * QPC + j) * N

        def _zfill(i, _):
            r = i // VR
            rows_b[r, pl.ds((i % VR) * LANES, LANES)] = jnp.zeros(
                (LANES,), jnp.float32)
            return 0
        lax.fori_loop(0, G * VR, _zfill, 0)

        def _zcopy(k, _):
            pltpu.sync_copy(rows_b, acc_sp.at[pl.ds(row0 + k * G, G)])
            return 0
        lax.fori_loop(0, nrows // G, _zcopy, 0)

        plsc.subcore_barrier()

        def _slab(sl, _):
            pltpu.sync_copy(src_hbm.at[s * NSLAB + sl], srcq_s)
            pltpu.sync_copy(dst_hbm.at[s * NSLAB + sl], dst_s)
            pltpu.sync_copy(norm_hbm.at[s * NSLAB + sl], norm_s)

            def _shift(r, _):
                for v in range(GV):
                    vs = pl.ds(v * LANES, LANES)
                    srcq_s[r, vs] = srcq_s[r, vs] + qbase
                return 0

            lax.fori_loop(0, SLAB, _shift, 0)

            def issue_gather(g, k):
                pltpu.async_copy(x_hbm.at[srcq_s.at[g]], rows[k], gsem[k])

            def wait_gather(g, k):
                pltpu.make_async_copy(x_hbm.at[srcq_s.at[g]], rows[k],
                                      gsem[k]).wait()

            def issue_scatter(g, k):
                pltpu.async_copy(rows[k], acc_sp.at[dst_s.at[g]], ssem[k],
                                 add=True)

            def wait_scatter(g, k):
                pltpu.make_async_copy(rows[k], acc_sp.at[dst_s.at[g]],
                                      ssem[k]).wait()

            def scale(g, k):
                gvec = jnp.full((LANES,), g, jnp.int32)

                def _srow(i4, _):
                    for u in range(4):
                        i = i4 * 4 + u
                        nv = plsc.load_gather(
                            norm_s, [gvec, jnp.full((LANES,), i, jnp.int32)])
                        for v in range(VR):
                            vs = pl.ds(v * LANES, LANES)
                            rows[k][i, vs] = rows[k][i, vs] * nv
                    return 0

                lax.fori_loop(0, G // 4, _srow, 0)

            def step(g, k, issue_next, wait_prev):
                if issue_next:
                    if wait_prev:
                        wait_scatter(g - 1, 1 - k)
                    issue_gather(g + 1, 1 - k)
                wait_gather(g, k)
                scale(g, k)
                issue_scatter(g, k)

            g0 = jnp.int32(0)
            issue_gather(g0, 0)
            step(g0, 0, True, False)

            def _pair(i, _):
                step(2 * i + 1, 1, True, True)
                step(2 * i + 2, 0, True, True)
                return 0

            lax.fori_loop(0, (SLAB - 2) // 2, _pair, 0)

            glast = jnp.int32(SLAB - 1)
            step(glast, 1, False, False)
            wait_scatter(jnp.int32(SLAB - 2), 0)
            wait_scatter(glast, 1)
            return 0

        lax.fori_loop(0, NSLAB, _slab, 0)

        def _selfg(g, _):
            pltpu.sync_copy(x_hbm.at[pl.ds(qbase + row0 + g * G, G)], rows_b)

            gvec = jnp.full((LANES,), g, jnp.int32)

            def _srow(i4, _):
                for u in range(4):
                    i = i4 * 4 + u
                    nv = plsc.load_gather(
                        selfn_b, [gvec, jnp.full((LANES,), i, jnp.int32)])
                    for v in range(VR):
                        sl = pl.ds(v * LANES, LANES)
                        rows_b[i, sl] = rows_b[i, sl] * nv
                return 0

            lax.fori_loop(0, G // 4, _srow, 0)
            pltpu.sync_copy(rows_b, acc_sp.at[selfid_b.at[g]], add=True)
            return 0

        lax.fori_loop(0, nselfg, _selfg, 0)

        plsc.subcore_barrier()

        q = c * QPC + j

        @pl.when(s < NS - 1)
        def _():
            pltpu.sync_copy(acc_sp.at[pl.ds(row0, 640)],
                            agg_hbm.at[q, pl.ds(row0, 640)])

        @pl.when(s == NS - 1)
        def _():
            pltpu.sync_copy(acc_sp.at[pl.ds(9600, 400)],
                            agg_hbm.at[q, pl.ds(9600, 400)])


def _dense_kernel(agg_ref, Az, Ar, Ah, bzf, brf, bhf, Lz, Lr, Lh,
                  lw, lb, out_ref):
    f32 = jnp.float32

    def mm(a, w):
        return jax.lax.dot_general(a, w, (((1,), (0,)), ((), ())),
                                   preferred_element_type=f32)

    a_all = agg_ref[0].reshape(T * NB, FIN)
    Pz = (mm(a_all, Az[...]) + bzf[...]).reshape(T, NB, HID)
    Pr = (mm(a_all, Ar[...]) + brf[...]).reshape(T, NB, HID)
    Ph = (mm(a_all, Ah[...]) + bhf[...]).reshape(T, NB, HID)

    H = jnp.zeros((NB, HID), f32)
    hs = []
    for t in range(T):
        Z = jax.nn.sigmoid(Pz[t] + mm(H, Lz[...]))
        R = jax.nn.sigmoid(Pr[t] + mm(H, Lr[...]))
        Ht = jnp.tanh(Ph[t] + mm(H * R, Lh[...]))
        H = Z * H + (1.0 - Z) * Ht
        hs.append(jax.nn.relu(H))
    h_all = jnp.concatenate(hs, axis=0)
    out = mm(h_all, lw[...]) + lb[...]
    out_ref[0] = out.reshape(T, NB, OUT)


def kernel(x, edge_index, edge_weight, Wz, bz, Wr, br, Wh, bh,
           lz_w, lz_b, lr_w, lr_b, lh_w, lh_b, lin_w, lin_b):
    f32 = jnp.float32
    mesh = plsc.VectorSubcoreMesh(core_axis_name="c", subcore_axis_name="s",
                                  num_cores=NC, num_subcores=NS)
    sc_params = pltpu.CompilerParams(needs_layout_passes=False)

    src = edge_index[0].astype(jnp.int32).reshape(NS * NSLAB, SLAB, G)
    dst = edge_index[1].astype(jnp.int32).reshape(NS * NSLAB, SLAB, G)
    w_r = edge_weight.astype(f32).reshape(NS * NSLAB, SLAB, G)

    k1 = pl.kernel(
        _norm_kernel,
        out_type=(
            jax.ShapeDtypeStruct((NS * NSLAB, SLAB, G), f32),
            jax.ShapeDtypeStruct((N // G // SSLAB, SSLAB, G), f32),
        ),
        mesh=mesh,
        scratch_types=(
            pltpu.VMEM((SLAB, G), jnp.int32),
            pltpu.VMEM((SLAB, G), f32),
            pltpu.VMEM((SLAB, G), jnp.int32),
            pltpu.VMEM((N,), f32),
            pltpu.VMEM((SSLAB, G), f32),
            pltpu.VMEM((640,), f32),
            pltpu.VMEM_SHARED((N,), f32),
        ),
        compiler_params=sc_params,
    )
    norm_r, selfn_r = k1(src, dst, w_r)

    x_bt = jnp.transpose(x, (0, 3, 1, 2)).reshape(B * T * N, FIN)
    selfn_p = jnp.pad(selfn_r.reshape(N // G, G), ((0, 3), (0, 0)))
    selfid = jnp.minimum(jnp.arange(NS * 8 * G, dtype=jnp.int32),
                         N - 1).reshape(NS * 8, G)

    k2 = pl.kernel(
        _prop_kernel,
        out_type=jax.ShapeDtypeStruct((B * T, N, FIN), f32),
        mesh=mesh,
        scratch_types=(
            pltpu.VMEM((SLAB, G), jnp.int32),
            pltpu.VMEM((SLAB, G), jnp.int32),
            pltpu.VMEM((SLAB, G), f32),
            pltpu.VMEM((8, G), f32),
            pltpu.VMEM((8, G), jnp.int32),
            pltpu.VMEM((G, FIN), f32),
            pltpu.VMEM((G, FIN), f32),
            pltpu.SemaphoreType.DMA,
            pltpu.SemaphoreType.DMA,
            pltpu.SemaphoreType.DMA,
            pltpu.SemaphoreType.DMA,
            pltpu.VMEM_SHARED((N, FIN), f32),
        ),
        compiler_params=sc_params,
    )
    agg = k2(x_bt, src, dst, norm_r, selfn_p, selfid)
    agg4 = agg.reshape(B, T, N, FIN)

    def wspec(a):
        return pl.BlockSpec(a.shape, lambda b, i, _n=a.ndim: (0,) * _n)

    Az = Wz @ lz_w[:HID]
    Ar = Wr @ lr_w[:HID]
    Ah = Wh @ lh_w[:HID]
    bzf = (bz @ lz_w[:HID] + lz_b).reshape(1, HID)
    brf = (br @ lr_w[:HID] + lr_b).reshape(1, HID)
    bhf = (bh @ lh_w[:HID] + lh_b).reshape(1, HID)
    lb2 = lin_b.reshape(1, OUT)
    wargs = (Az, Ar, Ah, bzf, brf, bhf,
             lz_w[HID:], lr_w[HID:], lh_w[HID:], lin_w, lb2)

    out = pl.pallas_call(
        _dense_kernel,
        grid=(B, N // NB),
        in_specs=[pl.BlockSpec((1, T, NB, FIN), lambda b, i: (b, 0, i, 0))]
                 + [wspec(a) for a in wargs],
        out_specs=pl.BlockSpec((1, T, NB, OUT), lambda b, i: (b, 0, i, 0)),
        out_shape=jax.ShapeDtypeStruct((B, T, N, OUT), f32),
    )(agg4, *wargs)
    return out
